# Initial kernel scaffold; baseline (speedup 1.0000x reference)
import jax
import jax.numpy as jnp
from jax.experimental import pallas as pl


def kernel(x_now, sat_type, sage0_Wl, sage0_bl, sage0_Wr, sage1_Wl, sage1_bl, sage1_Wr, bn_gamma, bn_beta, W_out, b_out):
    raise NotImplementedError("write your pallas kernel here")


# speedup vs baseline: 903.6981x; 903.6981x over previous
import functools

import jax
import jax.numpy as jnp
from jax import lax
from jax.experimental import pallas as pl

N = 2048
H = 128
BM = 256
SIM_T = 0.9
_HI = lax.Precision.HIGHEST


def _sage_body(h_ref, x0_ref, satc_ref, satr_ref, wl_ref, bl_ref, wr_ref,
               out_ref):
    i = pl.program_id(0)

    def _fn(x):
        lane = lax.broadcasted_iota(jnp.int32, x.shape, 1)
        xm = jnp.where(lane < 4, x, 0.0)
        ns = jnp.sum(xm * xm, axis=1, keepdims=True)
        return xm / jnp.maximum(jnp.sqrt(ns), 1e-12)

    fn = _fn(x0_ref[...])
    fblk = _fn(x0_ref[pl.ds(i * BM, BM), :])
    cos = lax.dot_general(fblk, fn, (((1,), (1,)), ((), ())))

    satc = satc_ref[pl.ds(i * BM, BM), :]
    same = satc == satr_ref[...]
    row = i * BM + lax.broadcasted_iota(jnp.int32, (BM, N), 0)
    col = lax.broadcasted_iota(jnp.int32, (BM, N), 1)
    keep = (same | (cos > SIM_T)) & (row != col)
    a = jnp.where(keep, 1.0, 0.0)

    deg = jnp.sum(a, axis=1, keepdims=True)
    h = h_ref[...]
    agg = lax.dot_general(a, h, (((1,), (0,)), ((), ())), precision=_HI)
    agg = agg / jnp.maximum(deg, 1.0)

    hblk = h_ref[pl.ds(i * BM, BM), :]
    z = (lax.dot_general(agg, wl_ref[...], (((1,), (1,)), ((), ())))
         + bl_ref[...]
         + lax.dot_general(hblk, wr_ref[...], (((1,), (1,)), ((), ()))))
    out_ref[...] = jnp.maximum(z, 0.0)


def _bn_out_body(h_ref, g_ref, b_ref, w_ref, bo_ref, hbn_ref, out_ref):
    h = h_ref[...]
    mean = jnp.mean(h, axis=0, keepdims=True)
    var = jnp.mean((h - mean) ** 2, axis=0, keepdims=True)
    hbn = (h - mean) / jnp.sqrt(var + 1e-5) * g_ref[...] + b_ref[...]
    hbn_ref[...] = hbn
    out_ref[...] = (lax.dot_general(hbn, w_ref[...], (((1,), (1,)), ((), ())))
                    + bo_ref[...])


_full = lambda shape: pl.BlockSpec(shape, lambda i: (0,) * len(shape))


def _sage_layer(h, x0, satc, satr, wl, bl, wr):
    return pl.pallas_call(
        _sage_body,
        grid=(N // BM,),
        in_specs=[_full((N, H)), _full((N, H)), _full((N, 1)),
                  _full((1, N)), _full((H, H)), _full((1, H)),
                  _full((H, H))],
        out_specs=pl.BlockSpec((BM, H), lambda i: (i, 0)),
        out_shape=jax.ShapeDtypeStruct((N, H), jnp.float32),
    )(h, x0, satc, satr, wl, bl, wr)


def kernel(x_now, sat_type, sage0_Wl, sage0_bl, sage0_Wr, sage1_Wl, sage1_bl,
           sage1_Wr, bn_gamma, bn_beta, W_out, b_out):
    h0 = jnp.concatenate([x_now[:, :1], x_now], axis=1)
    sat = sat_type.astype(jnp.int32)
    satc = sat.reshape(N, 1)
    satr = sat.reshape(1, N)
    bl0 = sage0_bl.reshape(1, H)
    bl1 = sage1_bl.reshape(1, H)

    h1 = _sage_layer(h0, h0, satc, satr, sage0_Wl, bl0, sage0_Wr)
    h2 = _sage_layer(h1, h0, satc, satr, sage1_Wl, bl1, sage1_Wr)

    w_pad = jnp.zeros((H, H), jnp.float32).at[:W_out.shape[0]].set(W_out)
    b_pad = jnp.zeros((1, H), jnp.float32).at[0, :b_out.shape[0]].set(b_out)
    hbn, out_pad = pl.pallas_call(
        _bn_out_body,
        grid=(1,),
        in_specs=[_full((N, H)), _full((1, H)), _full((1, H)),
                  _full((H, H)), _full((1, H))],
        out_specs=[pl.BlockSpec((N, H), lambda i: (0, 0)),
                   pl.BlockSpec((N, H), lambda i: (0, 0))],
        out_shape=[jax.ShapeDtypeStruct((N, H), jnp.float32),
                   jax.ShapeDtypeStruct((N, H), jnp.float32)],
    )(h2, bn_gamma.reshape(1, H), bn_beta.reshape(1, H), w_pad, b_pad)
    return (hbn, out_pad[:, :W_out.shape[0]])


# speedup vs baseline: 1490.1593x; 1.6490x over previous
import jax
import jax.numpy as jnp
from jax import lax
from jax.experimental import pallas as pl
from jax.experimental.pallas import tpu as pltpu

N = 2048
H = 128
BM = 256
SIM_T = 0.9
_DN = (((1,), (1,)), ((), ()))


def _sage_body(h_ref, x0_ref, satc_ref, satr_ref, wl_ref, bl_ref, wr_ref,
               out_ref, fn_ref, hhi_ref, hlo_ref):
    i = pl.program_id(0)

    @pl.when(i == 0)
    def _prologue():
        x0 = x0_ref[...]
        lane = lax.broadcasted_iota(jnp.int32, x0.shape, 1)
        xm = jnp.where(lane < 4, x0, 0.0)
        ns = jnp.sum(xm * xm, axis=1, keepdims=True)
        fn_ref[...] = xm / jnp.maximum(jnp.sqrt(ns), 1e-12)
        h = h_ref[...]
        hhi = h.astype(jnp.bfloat16)
        hhi_ref[...] = hhi
        hlo_ref[...] = (h - hhi.astype(jnp.float32)).astype(jnp.bfloat16)

    fn = fn_ref[...]
    fblk = fn_ref[pl.ds(i * BM, BM), :]
    cos = lax.dot_general(fblk, fn, _DN)

    satc = satc_ref[pl.ds(i * BM, BM), :]
    keep = (satc == satr_ref[...]) | (cos > SIM_T)
    af = jnp.where(keep, 1.0, 0.0)
    a = af.astype(jnp.bfloat16)

    deg = jnp.sum(af, axis=1, keepdims=True) - 1.0
    agg = (lax.dot_general(a, hhi_ref[...], (((1,), (0,)), ((), ())),
                           preferred_element_type=jnp.float32)
           + lax.dot_general(a, hlo_ref[...], (((1,), (0,)), ((), ())),
                             preferred_element_type=jnp.float32))
    hblk = h_ref[pl.ds(i * BM, BM), :]
    agg = (agg - hblk) / jnp.maximum(deg, 1.0)

    z = (lax.dot_general(agg, wl_ref[...], _DN)
         + bl_ref[...]
         + lax.dot_general(hblk, wr_ref[...], _DN))
    out_ref[...] = jnp.maximum(z, 0.0)


def _bn_out_body(h_ref, g_ref, b_ref, w_ref, bo_ref, hbn_ref, out_ref):
    h = h_ref[...]
    mean = jnp.mean(h, axis=0, keepdims=True)
    var = jnp.mean((h - mean) ** 2, axis=0, keepdims=True)
    hbn = (h - mean) / jnp.sqrt(var + 1e-5) * g_ref[...] + b_ref[...]
    hbn_ref[...] = hbn
    out_ref[...] = lax.dot_general(hbn, w_ref[...], _DN) + bo_ref[...]


_full = lambda shape: pl.BlockSpec(shape, lambda i: (0,) * len(shape))


def _sage_layer(h, x0, satc, satr, wl, bl, wr):
    return pl.pallas_call(
        _sage_body,
        grid=(N // BM,),
        in_specs=[_full((N, H)), _full((N, H)), _full((N, 1)),
                  _full((1, N)), _full((H, H)), _full((1, H)),
                  _full((H, H))],
        out_specs=pl.BlockSpec((BM, H), lambda i: (i, 0)),
        out_shape=jax.ShapeDtypeStruct((N, H), jnp.float32),
        scratch_shapes=[pltpu.VMEM((N, H), jnp.float32),
                        pltpu.VMEM((N, H), jnp.bfloat16),
                        pltpu.VMEM((N, H), jnp.bfloat16)],
    )(h, x0, satc, satr, wl, bl, wr)


def kernel(x_now, sat_type, sage0_Wl, sage0_bl, sage0_Wr, sage1_Wl, sage1_bl,
           sage1_Wr, bn_gamma, bn_beta, W_out, b_out):
    h0 = jnp.concatenate([x_now[:, :1], x_now], axis=1)
    sat = sat_type.astype(jnp.int32)
    satc = sat.reshape(N, 1)
    satr = sat.reshape(1, N)
    bl0 = sage0_bl.reshape(1, H)
    bl1 = sage1_bl.reshape(1, H)

    h1 = _sage_layer(h0, h0, satc, satr, sage0_Wl, bl0, sage0_Wr)
    h2 = _sage_layer(h1, h0, satc, satr, sage1_Wl, bl1, sage1_Wr)

    w_pad = jnp.zeros((H, H), jnp.float32).at[:W_out.shape[0]].set(W_out)
    b_pad = jnp.zeros((1, H), jnp.float32).at[0, :b_out.shape[0]].set(b_out)
    hbn, out_pad = pl.pallas_call(
        _bn_out_body,
        grid=(1,),
        in_specs=[_full((N, H)), _full((1, H)), _full((1, H)),
                  _full((H, H)), _full((1, H))],
        out_specs=[pl.BlockSpec((N, H), lambda i: (0, 0)),
                   pl.BlockSpec((N, H), lambda i: (0, 0))],
        out_shape=[jax.ShapeDtypeStruct((N, H), jnp.float32),
                   jax.ShapeDtypeStruct((N, H), jnp.float32)],
    )(h2, bn_gamma.reshape(1, H), bn_beta.reshape(1, H), w_pad, b_pad)
    return (hbn, out_pad[:, :W_out.shape[0]])


# speedup vs baseline: 1918.3337x; 1.2873x over previous
import jax
import jax.numpy as jnp
from jax import lax
from jax.experimental import pallas as pl
from jax.experimental.pallas import tpu as pltpu

N = 2048
H = 128
BM = 256
G = N // BM
SIM_T = 0.9
_DN = (((1,), (1,)), ((), ()))
_AH = (((1,), (0,)), ((), ()))


def _fused_body(h0_ref, satc_ref, satr_ref, wl0_ref, bl0_ref, wr0_ref,
                wl1_ref, bl1_ref, wr1_ref, g_ref, b_ref, wo_ref, bo_ref,
                hbn_ref, out_ref,
                fn_ref, hhi_ref, hlo_ref, am_ref, deg_ref, h1_ref, h2_ref):
    s = pl.program_id(0)

    def split_h(src_ref):
        h = src_ref[...]
        hhi = h.astype(jnp.bfloat16)
        hhi_ref[...] = hhi
        hlo_ref[...] = (h - hhi.astype(jnp.float32)).astype(jnp.bfloat16)

    def sage_block(i, hsrc_ref, wl_ref, bl_ref, wr_ref, dst_ref):
        a = am_ref[pl.ds(i * BM, BM), :]
        deg = deg_ref[pl.ds(i * BM, BM), :]
        agg = (lax.dot_general(a, hhi_ref[...], _AH,
                               preferred_element_type=jnp.float32)
               + lax.dot_general(a, hlo_ref[...], _AH,
                                 preferred_element_type=jnp.float32))
        hblk = hsrc_ref[pl.ds(i * BM, BM), :]
        agg = (agg - hblk) / jnp.maximum(deg, 1.0)
        z = (lax.dot_general(agg, wl_ref[...], _DN)
             + bl_ref[...]
             + lax.dot_general(hblk, wr_ref[...], _DN))
        dst_ref[pl.ds(i * BM, BM), :] = jnp.maximum(z, 0.0)

    @pl.when(s == 0)
    def _prologue():
        x0 = h0_ref[...]
        lane = lax.broadcasted_iota(jnp.int32, x0.shape, 1)
        xm = jnp.where(lane < 4, x0, 0.0)
        ns = jnp.sum(xm * xm, axis=1, keepdims=True)
        fn_ref[...] = xm / jnp.maximum(jnp.sqrt(ns), 1e-12)
        split_h(h0_ref)

    @pl.when(s < G)
    def _layer0():
        i = s
        fblk = fn_ref[pl.ds(i * BM, BM), :]
        cos = lax.dot_general(fblk, fn_ref[...], _DN)
        satc = satc_ref[pl.ds(i * BM, BM), :]
        keep = (satc == satr_ref[...]) | (cos > SIM_T)
        af = jnp.where(keep, 1.0, 0.0)
        am_ref[pl.ds(i * BM, BM), :] = af.astype(jnp.bfloat16)
        deg_ref[pl.ds(i * BM, BM), :] = (
            jnp.sum(af, axis=1, keepdims=True) - 1.0)
        sage_block(i, h0_ref, wl0_ref, bl0_ref, wr0_ref, h1_ref)

    @pl.when(s == G)
    def _relsplit():
        split_h(h1_ref)

    @pl.when((s >= G) & (s < 2 * G))
    def _layer1():
        sage_block(s - G, h1_ref, wl1_ref, bl1_ref, wr1_ref, h2_ref)

    @pl.when(s == 2 * G)
    def _bn_out():
        h = h2_ref[...]
        mean = jnp.mean(h, axis=0, keepdims=True)
        var = jnp.mean((h - mean) ** 2, axis=0, keepdims=True)
        hbn = (h - mean) / jnp.sqrt(var + 1e-5) * g_ref[...] + b_ref[...]
        hbn_ref[...] = hbn
        out_ref[...] = lax.dot_general(hbn, wo_ref[...], _DN) + bo_ref[...]


_full = lambda shape: pl.BlockSpec(shape, lambda i: (0,) * len(shape))


def kernel(x_now, sat_type, sage0_Wl, sage0_bl, sage0_Wr, sage1_Wl, sage1_bl,
           sage1_Wr, bn_gamma, bn_beta, W_out, b_out):
    h0 = jnp.concatenate([x_now[:, :1], x_now], axis=1)
    sat = sat_type.astype(jnp.int32)
    w_pad = jnp.zeros((H, H), jnp.float32).at[:W_out.shape[0]].set(W_out)
    b_pad = jnp.zeros((1, H), jnp.float32).at[0, :b_out.shape[0]].set(b_out)

    hbn, out_pad = pl.pallas_call(
        _fused_body,
        grid=(2 * G + 1,),
        in_specs=[_full((N, H)), _full((N, 1)), _full((1, N)),
                  _full((H, H)), _full((1, H)), _full((H, H)),
                  _full((H, H)), _full((1, H)), _full((H, H)),
                  _full((1, H)), _full((1, H)), _full((H, H)),
                  _full((1, H))],
        out_specs=[pl.BlockSpec((N, H), lambda i: (0, 0)),
                   pl.BlockSpec((N, H), lambda i: (0, 0))],
        out_shape=[jax.ShapeDtypeStruct((N, H), jnp.float32),
                   jax.ShapeDtypeStruct((N, H), jnp.float32)],
        scratch_shapes=[pltpu.VMEM((N, H), jnp.float32),
                        pltpu.VMEM((N, H), jnp.bfloat16),
                        pltpu.VMEM((N, H), jnp.bfloat16),
                        pltpu.VMEM((N, N), jnp.bfloat16),
                        pltpu.VMEM((N, 1), jnp.float32),
                        pltpu.VMEM((N, H), jnp.float32),
                        pltpu.VMEM((N, H), jnp.float32)],
    )(h0, sat.reshape(N, 1), sat.reshape(1, N),
      sage0_Wl, sage0_bl.reshape(1, H), sage0_Wr,
      sage1_Wl, sage1_bl.reshape(1, H), sage1_Wr,
      bn_gamma.reshape(1, H), bn_beta.reshape(1, H), w_pad, b_pad)
    return (hbn, out_pad[:, :W_out.shape[0]])


# speedup vs baseline: 1985.9569x; 1.0353x over previous
import jax
import jax.numpy as jnp
from jax import lax
from jax.experimental import pallas as pl
from jax.experimental.pallas import tpu as pltpu

N = 2048
H = 128
BM = 512
G = N // BM
SIM_T = 0.9
_DN = (((1,), (1,)), ((), ()))
_AH = (((1,), (0,)), ((), ()))


def _fused_body(h0_ref, satc_ref, satr_ref, wl0_ref, bl0_ref, wr0_ref,
                wl1_ref, bl1_ref, wr1_ref, g_ref, b_ref, wo_ref, bo_ref,
                hbn_ref, out_ref,
                fn_ref, hhi_ref, hlo_ref, am_ref, deg_ref, h1_ref, h2_ref):
    s = pl.program_id(0)

    def split_h(src_ref):
        h = src_ref[...]
        hhi = h.astype(jnp.bfloat16)
        hhi_ref[...] = hhi
        hlo_ref[...] = (h - hhi.astype(jnp.float32)).astype(jnp.bfloat16)

    def sage_block(i, hsrc_ref, wl_ref, bl_ref, wr_ref, dst_ref):
        a = am_ref[pl.ds(i * BM, BM), :]
        deg = deg_ref[pl.ds(i * BM, BM), :]
        agg = (lax.dot_general(a, hhi_ref[...], _AH,
                               preferred_element_type=jnp.float32)
               + lax.dot_general(a, hlo_ref[...], _AH,
                                 preferred_element_type=jnp.float32))
        hblk = hsrc_ref[pl.ds(i * BM, BM), :]
        agg = (agg - hblk) / jnp.maximum(deg, 1.0)
        z = (lax.dot_general(agg, wl_ref[...], _DN)
             + bl_ref[...]
             + lax.dot_general(hblk, wr_ref[...], _DN))
        dst_ref[pl.ds(i * BM, BM), :] = jnp.maximum(z, 0.0)

    @pl.when(s == 0)
    def _prologue():
        x0 = h0_ref[...]
        lane = lax.broadcasted_iota(jnp.int32, x0.shape, 1)
        xm = jnp.where(lane < 4, x0, 0.0)
        ns = jnp.sum(xm * xm, axis=1, keepdims=True)
        fn_ref[...] = xm / jnp.maximum(jnp.sqrt(ns), 1e-12)
        split_h(h0_ref)

    @pl.when(s < G)
    def _layer0():
        i = s
        fblk = fn_ref[pl.ds(i * BM, BM), :]
        cos = lax.dot_general(fblk, fn_ref[...], _DN)
        satc = satc_ref[pl.ds(i * BM, BM), :]
        keep = (satc == satr_ref[...]) | (cos > SIM_T)
        af = jnp.where(keep, 1.0, 0.0)
        am_ref[pl.ds(i * BM, BM), :] = af.astype(jnp.bfloat16)
        deg_ref[pl.ds(i * BM, BM), :] = (
            jnp.sum(af, axis=1, keepdims=True) - 1.0)
        sage_block(i, h0_ref, wl0_ref, bl0_ref, wr0_ref, h1_ref)

    @pl.when(s == G)
    def _relsplit():
        split_h(h1_ref)

    @pl.when((s >= G) & (s < 2 * G))
    def _layer1():
        sage_block(s - G, h1_ref, wl1_ref, bl1_ref, wr1_ref, h2_ref)

    @pl.when(s == 2 * G)
    def _bn_out():
        h = h2_ref[...]
        mean = jnp.mean(h, axis=0, keepdims=True)
        var = jnp.mean((h - mean) ** 2, axis=0, keepdims=True)
        hbn = (h - mean) / jnp.sqrt(var + 1e-5) * g_ref[...] + b_ref[...]
        hbn_ref[...] = hbn
        out_ref[...] = lax.dot_general(hbn, wo_ref[...], _DN) + bo_ref[...]


_full = lambda shape: pl.BlockSpec(shape, lambda i: (0,) * len(shape))


def kernel(x_now, sat_type, sage0_Wl, sage0_bl, sage0_Wr, sage1_Wl, sage1_bl,
           sage1_Wr, bn_gamma, bn_beta, W_out, b_out):
    h0 = jnp.concatenate([x_now[:, :1], x_now], axis=1)
    sat = sat_type.astype(jnp.int32)
    w_pad = jnp.zeros((H, H), jnp.float32).at[:W_out.shape[0]].set(W_out)
    b_pad = jnp.zeros((1, H), jnp.float32).at[0, :b_out.shape[0]].set(b_out)

    hbn, out_pad = pl.pallas_call(
        _fused_body,
        grid=(2 * G + 1,),
        in_specs=[_full((N, H)), _full((N, 1)), _full((1, N)),
                  _full((H, H)), _full((1, H)), _full((H, H)),
                  _full((H, H)), _full((1, H)), _full((H, H)),
                  _full((1, H)), _full((1, H)), _full((H, H)),
                  _full((1, H))],
        out_specs=[pl.BlockSpec((N, H), lambda i: (0, 0)),
                   pl.BlockSpec((N, H), lambda i: (0, 0))],
        out_shape=[jax.ShapeDtypeStruct((N, H), jnp.float32),
                   jax.ShapeDtypeStruct((N, H), jnp.float32)],
        scratch_shapes=[pltpu.VMEM((N, H), jnp.float32),
                        pltpu.VMEM((N, H), jnp.bfloat16),
                        pltpu.VMEM((N, H), jnp.bfloat16),
                        pltpu.VMEM((N, N), jnp.bfloat16),
                        pltpu.VMEM((N, 1), jnp.float32),
                        pltpu.VMEM((N, H), jnp.float32),
                        pltpu.VMEM((N, H), jnp.float32)],
    )(h0, sat.reshape(N, 1), sat.reshape(1, N),
      sage0_Wl, sage0_bl.reshape(1, H), sage0_Wr,
      sage1_Wl, sage1_bl.reshape(1, H), sage1_Wr,
      bn_gamma.reshape(1, H), bn_beta.reshape(1, H), w_pad, b_pad)
    return (hbn, out_pad[:, :W_out.shape[0]])


# speedup vs baseline: 2087.4844x; 1.0511x over previous
import jax
import jax.numpy as jnp
from jax import lax
from jax.experimental import pallas as pl
from jax.experimental.pallas import tpu as pltpu

N = 2048
H = 128
BM = 1024
G = N // BM
SIM_T = 0.9
_DN = (((1,), (1,)), ((), ()))
_AH = (((1,), (0,)), ((), ()))


def _fused_body(h0_ref, satc_ref, satr_ref, wl0_ref, bl0_ref, wr0_ref,
                wl1_ref, bl1_ref, wr1_ref, g_ref, b_ref, wo_ref, bo_ref,
                hbn_ref, out_ref,
                fn_ref, hhi_ref, hlo_ref, am_ref, deg_ref, h1_ref, h2_ref):
    s = pl.program_id(0)

    def split_h(src_ref):
        h = src_ref[...]
        hhi = h.astype(jnp.bfloat16)
        hhi_ref[...] = hhi
        hlo_ref[...] = (h - hhi.astype(jnp.float32)).astype(jnp.bfloat16)

    def sage_block(i, hsrc_ref, wl_ref, bl_ref, wr_ref, dst_ref):
        a = am_ref[pl.ds(i * BM, BM), :]
        deg = deg_ref[pl.ds(i * BM, BM), :]
        agg = (lax.dot_general(a, hhi_ref[...], _AH,
                               preferred_element_type=jnp.float32)
               + lax.dot_general(a, hlo_ref[...], _AH,
                                 preferred_element_type=jnp.float32))
        hblk = hsrc_ref[pl.ds(i * BM, BM), :]
        agg = (agg - hblk) / jnp.maximum(deg, 1.0)
        z = (lax.dot_general(agg, wl_ref[...], _DN)
             + bl_ref[...]
             + lax.dot_general(hblk, wr_ref[...], _DN))
        dst_ref[pl.ds(i * BM, BM), :] = jnp.maximum(z, 0.0)

    @pl.when(s == 0)
    def _prologue():
        x0 = h0_ref[...]
        lane = lax.broadcasted_iota(jnp.int32, x0.shape, 1)
        xm = jnp.where(lane < 4, x0, 0.0)
        ns = jnp.sum(xm * xm, axis=1, keepdims=True)
        fn_ref[...] = xm / jnp.maximum(jnp.sqrt(ns), 1e-12)
        split_h(h0_ref)

    @pl.when(s < G)
    def _layer0():
        i = s
        fblk = fn_ref[pl.ds(i * BM, BM), :]
        cos = lax.dot_general(fblk, fn_ref[...], _DN)
        satc = satc_ref[pl.ds(i * BM, BM), :]
        keep = (satc == satr_ref[...]) | (cos > SIM_T)
        af = jnp.where(keep, 1.0, 0.0)
        am_ref[pl.ds(i * BM, BM), :] = af.astype(jnp.bfloat16)
        deg_ref[pl.ds(i * BM, BM), :] = (
            jnp.sum(af, axis=1, keepdims=True) - 1.0)
        sage_block(i, h0_ref, wl0_ref, bl0_ref, wr0_ref, h1_ref)

    @pl.when(s == G)
    def _relsplit():
        split_h(h1_ref)

    @pl.when((s >= G) & (s < 2 * G))
    def _layer1():
        sage_block(s - G, h1_ref, wl1_ref, bl1_ref, wr1_ref, h2_ref)

    @pl.when(s == 2 * G)
    def _bn_out():
        h = h2_ref[...]
        mean = jnp.mean(h, axis=0, keepdims=True)
        var = jnp.mean((h - mean) ** 2, axis=0, keepdims=True)
        hbn = (h - mean) / jnp.sqrt(var + 1e-5) * g_ref[...] + b_ref[...]
        hbn_ref[...] = hbn
        out_ref[...] = lax.dot_general(hbn, wo_ref[...], _DN) + bo_ref[...]


_full = lambda shape: pl.BlockSpec(shape, lambda i: (0,) * len(shape))


def kernel(x_now, sat_type, sage0_Wl, sage0_bl, sage0_Wr, sage1_Wl, sage1_bl,
           sage1_Wr, bn_gamma, bn_beta, W_out, b_out):
    h0 = jnp.concatenate([x_now[:, :1], x_now], axis=1)
    sat = sat_type.astype(jnp.int32)
    w_pad = jnp.zeros((H, H), jnp.float32).at[:W_out.shape[0]].set(W_out)
    b_pad = jnp.zeros((1, H), jnp.float32).at[0, :b_out.shape[0]].set(b_out)

    hbn, out_pad = pl.pallas_call(
        _fused_body,
        grid=(2 * G + 1,),
        in_specs=[_full((N, H)), _full((N, 1)), _full((1, N)),
                  _full((H, H)), _full((1, H)), _full((H, H)),
                  _full((H, H)), _full((1, H)), _full((H, H)),
                  _full((1, H)), _full((1, H)), _full((H, H)),
                  _full((1, H))],
        out_specs=[pl.BlockSpec((N, H), lambda i: (0, 0)),
                   pl.BlockSpec((N, H), lambda i: (0, 0))],
        out_shape=[jax.ShapeDtypeStruct((N, H), jnp.float32),
                   jax.ShapeDtypeStruct((N, H), jnp.float32)],
        scratch_shapes=[pltpu.VMEM((N, H), jnp.float32),
                        pltpu.VMEM((N, H), jnp.bfloat16),
                        pltpu.VMEM((N, H), jnp.bfloat16),
                        pltpu.VMEM((N, N), jnp.bfloat16),
                        pltpu.VMEM((N, 1), jnp.float32),
                        pltpu.VMEM((N, H), jnp.float32),
                        pltpu.VMEM((N, H), jnp.float32)],
    )(h0, sat.reshape(N, 1), sat.reshape(1, N),
      sage0_Wl, sage0_bl.reshape(1, H), sage0_Wr,
      sage1_Wl, sage1_bl.reshape(1, H), sage1_Wr,
      bn_gamma.reshape(1, H), bn_beta.reshape(1, H), w_pad, b_pad)
    return (hbn, out_pad[:, :W_out.shape[0]])
